# baseline (speedup 1.0000x reference)
import functools

import jax
import jax.numpy as jnp
from jax import lax
from jax.experimental import pallas as pl
from jax.experimental.pallas import tpu as pltpu
from jax.experimental.pallas import tpu_sc as plsc

_R = 2
_T = 200
_SPLITS = ((0, 104), (104, 96))


def _gather_body(
    rows_per_w, num_cores, tok_hbm, w_hbm, out_hbm, idx_v, rows_v, sem_idx, sem_g, sem_o
):
    n_chunks = rows_per_w // _R
    wid = lax.axis_index("s") * num_cores + lax.axis_index("c")
    row_base = wid * rows_per_w

    def idx_start(i, slot):
        pltpu.async_copy(
            tok_hbm.at[pl.ds(row_base + i * _R, _R)], idx_v.at[slot], sem_idx
        )

    def idx_wait(slot):
        pltpu.make_async_copy(
            tok_hbm.at[pl.ds(row_base, _R)], idx_v.at[slot], sem_idx
        ).wait()

    def gathers_start(slot):
        for r in range(_R):
            for off, n in _SPLITS:
                pltpu.async_copy(
                    w_hbm.at[idx_v.at[slot, r, pl.ds(off, n)]],
                    rows_v.at[slot, r].at[pl.ds(off, n)],
                    sem_g,
                )

    def gathers_wait(slot):
        for r in range(_R):
            for off, n in _SPLITS:
                pltpu.make_async_copy(
                    w_hbm.at[pl.ds(0, n)],
                    rows_v.at[slot, r].at[pl.ds(off, n)],
                    sem_g,
                ).wait()

    def out_start(i, slot):
        pltpu.async_copy(
            rows_v.at[slot, :, :, pl.ds(0, 64)],
            out_hbm.at[pl.ds(row_base + i * _R, _R)],
            sem_o,
        )

    def out_wait(slot):
        pltpu.make_async_copy(
            rows_v.at[slot, :, :, pl.ds(0, 64)],
            out_hbm.at[pl.ds(row_base, _R)],
            sem_o,
        ).wait()

    idx_start(0, 0)
    idx_wait(0)
    gathers_start(0)
    idx_start(1, 1)

    def loop_body(i, carry):
        slot = lax.rem(i, 2)
        nslot = lax.rem(i + 1, 2)
        gathers_wait(slot)
        out_start(i, slot)
        idx_wait(nslot)

        @pl.when(i >= 1)
        def _():
            out_wait(nslot)

        gathers_start(nslot)

        @pl.when(i + 2 < n_chunks)
        def _():
            idx_start(i + 2, slot)

        return carry

    lax.fori_loop(0, n_chunks - 1, loop_body, 0)

    last_slot = (n_chunks - 1) % 2
    gathers_wait(last_slot)
    out_start(n_chunks - 1, last_slot)
    out_wait(1 - last_slot)
    out_wait(last_slot)


def kernel(token_ids, W):
    B, T = token_ids.shape
    D = W.shape[1]
    w_pad = jnp.pad(W, ((0, 0), (0, 128 - D)))
    info = plsc.get_sparse_core_info()
    nw = info.num_cores * info.num_subcores
    rows_per_w = B // nw

    mesh = plsc.VectorSubcoreMesh(core_axis_name="c", subcore_axis_name="s")
    kfn = pl.kernel(
        functools.partial(_gather_body, rows_per_w, info.num_cores),
        out_type=jax.ShapeDtypeStruct((B, T, D), jnp.float32),
        mesh=mesh,
        scratch_types=[
            pltpu.VMEM((2, _R, _T), jnp.int32),
            pltpu.VMEM((2, _R, _T, 128), jnp.float32),
            pltpu.SemaphoreType.DMA,
            pltpu.SemaphoreType.DMA,
            pltpu.SemaphoreType.DMA,
        ],
        compiler_params=pltpu.CompilerParams(use_tc_tiling_on_sc=False),
    )
    return kfn(token_ids, w_pad)
